# baseline (speedup 1.0000x reference)
import functools

import jax
import jax.numpy as jnp
from jax import lax
from jax.experimental import pallas as pl
from jax.experimental.pallas import tpu as pltpu
from jax.experimental.pallas import tpu_sc as plsc

S, T = 4096, 200
D = 64
SCALE = 8.0
NC, NS, L = 2, 16, 16
NW = NC * NS
B = S * T
B_PER_W = B // NW
W = 16
NSEG = D // W
PSEG = 5
CHUNK = 128
N_CHUNK = B_PER_W // CHUNK
N_BUF = 8
LOOK = 3


def _emb_body(xf_hbm, tab_hbm, out_hbm, idx_v, bufs, gsems, wsems):
    wid = lax.axis_index("s") * NC + lax.axis_index("c")
    base = wid * B_PER_W

    pltpu.sync_copy(xf_hbm.at[pl.ds(base, B_PER_W)], idx_v)

    def start_gather(k, sl):
        pltpu.async_copy(
            tab_hbm.at[idx_v.at[pl.ds(k * CHUNK, CHUNK)]],
            bufs.at[sl], gsems.at[sl])

    def wait_gather(sl):
        pltpu.make_async_copy(
            tab_hbm.at[idx_v.at[pl.ds(0, CHUNK)]],
            bufs.at[0], gsems.at[sl]).wait()

    def start_write(k, sl):
        pltpu.async_copy(
            bufs.at[sl, :, pl.ds(0, NSEG)],
            out_hbm.at[pl.ds(base + k * CHUNK, CHUNK), pl.ds(0, NSEG)],
            wsems.at[sl])

    def wait_write(sl):
        pltpu.make_async_copy(
            bufs.at[0, :, pl.ds(0, NSEG)],
            out_hbm.at[pl.ds(0, CHUNK), pl.ds(0, NSEG)], wsems.at[sl]).wait()

    def scale_rows(sl):
        @pl.loop(0, CHUNK, unroll=8)
        def row_step(r):
            for j in range(NSEG):
                bufs[sl, r, j] = bufs[sl, r, j] * SCALE

    for k in range(LOOK):
        start_gather(k, k % N_BUF)

    @pl.loop(0, N_CHUNK, step=N_BUF)
    def outer(j0):
        for b in range(N_BUF):
            k = j0 + b
            bl = (b + LOOK) % N_BUF

            @pl.when(k + LOOK < N_CHUNK)
            def _():
                @pl.when(k + LOOK >= N_BUF)
                def _():
                    wait_write(bl)
                start_gather(k + LOOK, bl)

            wait_gather(b)
            scale_rows(b)
            start_write(k, b)

    for b in range(N_BUF):
        wait_write(b)


@jax.jit
def _emb_call(xf, tab5):
    return pl.kernel(
        _emb_body,
        out_type=jax.ShapeDtypeStruct((B, 2 * D // W, W), jnp.float32),
        mesh=plsc.VectorSubcoreMesh(core_axis_name="c", subcore_axis_name="s"),
        scratch_types=[
            pltpu.VMEM((B_PER_W,), jnp.int32),
            pltpu.VMEM((N_BUF, CHUNK, PSEG, W), jnp.float32),
            pltpu.SemaphoreType.DMA((N_BUF,)),
            pltpu.SemaphoreType.DMA((N_BUF,)),
        ],
        compiler_params=pltpu.CompilerParams(use_tc_tiling_on_sc=False),
    )(xf, tab5)


def kernel(x, table):
    xf = x.reshape(B)
    tpad = jnp.pad(table, ((0, 0), (0, PSEG * W - D)))
    tab5 = tpad.reshape(1000000, PSEG, W)
    out3 = _emb_call(xf, tab5)
    return out3.reshape(S, T, 2 * D)[:, :, :D]


# speedup vs baseline: 3.5809x; 3.5809x over previous
import functools

import jax
import jax.numpy as jnp
from jax import lax
from jax.experimental import pallas as pl
from jax.experimental.pallas import tpu as pltpu
from jax.experimental.pallas import tpu_sc as plsc

S, T = 4096, 200
D = 64
SCALE = 8.0
NC, NS, L = 2, 16, 16
NW = NC * NS
ROWS_PER_W = S // NW
B_PER_W = ROWS_PER_W * T
G0 = 128
G1 = T - G0
NVEC = 13
N_BUF = 8
LOOK = 3


def _emb_body(xf_hbm, tab_hbm, out_hbm, idx_v, bufs, gsems, wsems):
    wid = lax.axis_index("s") * NC + lax.axis_index("c")
    base = wid * ROWS_PER_W

    pltpu.sync_copy(xf_hbm.at[pl.ds(base * T, B_PER_W)], idx_v)

    def start_gather(r, sl):
        pltpu.async_copy(
            tab_hbm.at[idx_v.at[pl.ds(r * T, G0)]],
            bufs.at[sl, pl.ds(0, G0)], gsems.at[sl])
        pltpu.async_copy(
            tab_hbm.at[idx_v.at[pl.ds(r * T + G0, G1)]],
            bufs.at[sl, pl.ds(G0, G1)], gsems.at[sl])

    def wait_gather(sl):
        pltpu.make_async_copy(
            tab_hbm.at[idx_v.at[pl.ds(0, G0)]],
            bufs.at[0, pl.ds(0, G0)], gsems.at[sl]).wait()
        pltpu.make_async_copy(
            tab_hbm.at[idx_v.at[pl.ds(G0, G1)]],
            bufs.at[0, pl.ds(G0, G1)], gsems.at[sl]).wait()

    def start_write(r, sl):
        pltpu.async_copy(
            bufs.at[sl], out_hbm.at[base + r, :, pl.ds(0, D)], wsems.at[sl])

    def wait_write(sl):
        pltpu.make_async_copy(
            bufs.at[0], out_hbm.at[0, :, pl.ds(0, D)], wsems.at[sl]).wait()

    def scale_rows(sl):
        @pl.loop(0, T, unroll=8)
        def row_step(t):
            for c in range(D // L):
                csl = pl.ds(c * L, L)
                bufs[sl, t, csl] = bufs[sl, t, csl] * SCALE

    for r in range(LOOK):
        start_gather(r, r % N_BUF)

    @pl.loop(0, ROWS_PER_W, step=N_BUF)
    def outer(j0):
        for b in range(N_BUF):
            r = j0 + b
            bl = (b + LOOK) % N_BUF

            @pl.when(r + LOOK < ROWS_PER_W)
            def _():
                @pl.when(r + LOOK >= N_BUF)
                def _():
                    wait_write(bl)
                start_gather(r + LOOK, bl)

            wait_gather(b)
            scale_rows(b)
            start_write(r, b)

    for b in range(N_BUF):
        wait_write(b)


@jax.jit
def _emb_call(xf, tab2):
    return pl.kernel(
        _emb_body,
        out_type=jax.ShapeDtypeStruct((S, T, 2 * D), jnp.float32),
        mesh=plsc.VectorSubcoreMesh(core_axis_name="c", subcore_axis_name="s"),
        scratch_types=[
            pltpu.VMEM((B_PER_W,), jnp.int32),
            pltpu.VMEM((N_BUF, T, D), jnp.float32),
            pltpu.SemaphoreType.DMA((N_BUF,)),
            pltpu.SemaphoreType.DMA((N_BUF,)),
        ],
        compiler_params=pltpu.CompilerParams(use_tc_tiling_on_sc=False),
    )(xf, tab2)


def kernel(x, table):
    xf = x.reshape(S * T)
    out3 = _emb_call(xf, table)
    return out3[:, :, :D]


# speedup vs baseline: 3.8605x; 1.0781x over previous
import functools

import jax
import jax.numpy as jnp
from jax import lax
from jax.experimental import pallas as pl
from jax.experimental.pallas import tpu as pltpu
from jax.experimental.pallas import tpu_sc as plsc

S, T = 4096, 200
D = 64
SCALE = 8.0
NC, NS, L = 2, 16, 16
NW = NC * NS
ROWS_PER_W = S // NW
B_PER_W = ROWS_PER_W * T
G0 = 128
G1 = T - G0
NVEC = 13
N_BUF = 8
LOOK = 3


def _emb_body(xf_hbm, tab_hbm, out_hbm, idx_v, pidx_v, bufs, gsems, wsems):
    wid = lax.axis_index("s") * NC + lax.axis_index("c")
    base = wid * ROWS_PER_W

    pltpu.sync_copy(xf_hbm.at[pl.ds(base * T, B_PER_W)],
                    idx_v.at[pl.ds(0, B_PER_W)])

    def start_gather(r, sl):
        for c in range(NVEC):
            pidx_v[sl, pl.ds(c * L, L)] = (
                idx_v[pl.ds(r * T + c * L, L)] << 1)
        pltpu.async_copy(
            tab_hbm.at[pidx_v.at[sl, pl.ds(0, G0)]],
            bufs.at[sl, pl.ds(0, G0)], gsems.at[sl])
        pltpu.async_copy(
            tab_hbm.at[pidx_v.at[sl, pl.ds(G0, G1)]],
            bufs.at[sl, pl.ds(G0, G1)], gsems.at[sl])

    def wait_gather(sl):
        pltpu.make_async_copy(
            tab_hbm.at[pidx_v.at[0, pl.ds(0, G0)]],
            bufs.at[0, pl.ds(0, G0)], gsems.at[sl]).wait()
        pltpu.make_async_copy(
            tab_hbm.at[pidx_v.at[0, pl.ds(G0, G1)]],
            bufs.at[0, pl.ds(G0, G1)], gsems.at[sl]).wait()

    def start_write(r, sl):
        pltpu.async_copy(
            bufs.at[sl], out_hbm.at[base + r, :, pl.ds(0, D)], wsems.at[sl])

    def wait_write(sl):
        pltpu.make_async_copy(
            bufs.at[0], out_hbm.at[0, :, pl.ds(0, D)], wsems.at[sl]).wait()

    def scale_rows(sl):
        @pl.loop(0, T, unroll=8)
        def row_step(t):
            for c in range(D // L):
                csl = pl.ds(c * L, L)
                bufs[sl, t, csl] = bufs[sl, t, csl] * SCALE

    for r in range(LOOK):
        start_gather(r, r % N_BUF)

    @pl.loop(0, ROWS_PER_W, step=N_BUF)
    def outer(j0):
        for b in range(N_BUF):
            r = j0 + b
            bl = (b + LOOK) % N_BUF

            @pl.when(r + LOOK < ROWS_PER_W)
            def _():
                @pl.when(r + LOOK >= N_BUF)
                def _():
                    wait_write(bl)
                start_gather(r + LOOK, bl)

            wait_gather(b)
            scale_rows(b)
            start_write(r, b)

    for b in range(N_BUF):
        wait_write(b)


@jax.jit
def _emb_call(xf, tab2):
    return pl.kernel(
        _emb_body,
        out_type=jax.ShapeDtypeStruct((S, T, 2 * D), jnp.float32),
        mesh=plsc.VectorSubcoreMesh(core_axis_name="c", subcore_axis_name="s"),
        scratch_types=[
            pltpu.VMEM((B_PER_W + L,), jnp.int32),
            pltpu.VMEM((N_BUF, NVEC * L), jnp.int32),
            pltpu.VMEM((N_BUF, T, D), jnp.float32),
            pltpu.SemaphoreType.DMA((N_BUF,)),
            pltpu.SemaphoreType.DMA((N_BUF,)),
        ],
        compiler_params=pltpu.CompilerParams(use_tc_tiling_on_sc=False),
    )(xf, tab2)


def kernel(x, table):
    xf = x.reshape(S * T)
    tpad = jnp.pad(table, ((0, 0), (0, D)))
    tab2 = tpad.reshape(2 * 1000000, D)
    out3 = _emb_call(xf, tab2)
    return out3[:, :, :D]
